# baseline (speedup 1.0000x reference)
import jax
import jax.numpy as jnp
from jax.experimental import pallas as pl
from jax.experimental.pallas import tpu as pltpu

_DEPTH = 3


def _gc_kernel_body(adj_hbm, x_hbm, w_hbm, out_hbm,
                    x_buf, w_buf, adj_buf, o_buf,
                    adj_sem, x_sem, w_sem, out_sem,
                    *, B, W):
    bpc = B // 2
    npairs = bpc * W
    i = pl.program_id(0)
    b0 = i * bpc

    def adj_copy(k):
        b_local, w = divmod(k, W)
        return pltpu.make_async_copy(
            adj_hbm.at[b0 + b_local, w], adj_buf.at[k % _DEPTH],
            adj_sem.at[k % _DEPTH])

    def out_copy(k):
        b_local, w = divmod(k, W)
        return pltpu.make_async_copy(
            o_buf.at[k % 2], out_hbm.at[b0 + b_local, w], out_sem.at[k % 2])

    x_copy = pltpu.make_async_copy(x_hbm.at[pl.ds(b0, bpc)], x_buf, x_sem)
    w_copy = pltpu.make_async_copy(w_hbm, w_buf, w_sem)

    for k in range(min(_DEPTH, npairs)):
        adj_copy(k).start()
    x_copy.start()
    w_copy.start()
    x_copy.wait()
    w_copy.wait()

    for k in range(npairs):
        b_local, w = divmod(k, W)
        adj_copy(k).wait()
        if k >= 2:
            out_copy(k - 2).wait()
        xw = jnp.dot(x_buf[b_local, w], w_buf[w],
                     preferred_element_type=jnp.float32)
        o_buf[k % 2] = jnp.dot(adj_buf[k % _DEPTH], xw,
                               preferred_element_type=jnp.float32)
        out_copy(k).start()
        if k + _DEPTH < npairs:
            adj_copy(k + _DEPTH).start()

    out_copy(npairs - 2).wait()
    out_copy(npairs - 1).wait()


def kernel(adjacency, nodes, weights):
    B, W, N, _ = adjacency.shape
    Fin = nodes.shape[-1]
    Fout = weights.shape[-1]
    itemsize = jnp.dtype(adjacency.dtype).itemsize

    flops = 2 * B * W * (N * N * Fout + N * Fin * Fout)
    bytes_accessed = itemsize * (adjacency.size + nodes.size + weights.size
                                 + B * W * N * Fout)
    cost = pl.CostEstimate(flops=flops, transcendentals=0,
                           bytes_accessed=bytes_accessed)

    import functools
    body = functools.partial(_gc_kernel_body, B=B, W=W)

    return pl.pallas_call(
        body,
        out_shape=jax.ShapeDtypeStruct((B, W, N, Fout), nodes.dtype),
        grid=(2,),
        in_specs=[
            pl.BlockSpec(memory_space=pl.ANY),
            pl.BlockSpec(memory_space=pl.ANY),
            pl.BlockSpec(memory_space=pl.ANY),
        ],
        out_specs=pl.BlockSpec(memory_space=pl.ANY),
        scratch_shapes=[
            pltpu.VMEM((B // 2, W, N, Fin), jnp.float32),
            pltpu.VMEM((W, Fin, Fout), jnp.float32),
            pltpu.VMEM((_DEPTH, N, N), jnp.float32),
            pltpu.VMEM((2, N, Fout), jnp.float32),
            pltpu.SemaphoreType.DMA((_DEPTH,)),
            pltpu.SemaphoreType.DMA,
            pltpu.SemaphoreType.DMA,
            pltpu.SemaphoreType.DMA((2,)),
        ],
        compiler_params=pltpu.CompilerParams(
            dimension_semantics=("parallel",),
            vmem_limit_bytes=48 * 1024 * 1024,
        ),
        cost_estimate=cost,
    )(adjacency, nodes, weights)


# speedup vs baseline: 1.1959x; 1.1959x over previous
import jax
import jax.numpy as jnp
from jax.experimental import pallas as pl
from jax.experimental.pallas import tpu as pltpu


def _gc_kernel(adj_ref, x_ref, w_ref, out_ref):
    W = w_ref.shape[0]
    for w in range(W):
        xw = jnp.dot(x_ref[w], w_ref[w], preferred_element_type=jnp.float32)
        y = jnp.dot(adj_ref[w], xw, preferred_element_type=jnp.float32)
        out_ref[w] = y.astype(out_ref.dtype)


def kernel(adjacency, nodes, weights):
    B, W, N, _ = adjacency.shape
    Fin = nodes.shape[-1]
    Fout = weights.shape[-1]
    itemsize = jnp.dtype(adjacency.dtype).itemsize

    flops = 2 * B * W * (N * N * Fout + N * Fin * Fout)
    bytes_accessed = itemsize * (adjacency.size + nodes.size + weights.size
                                 + B * W * N * Fout)
    cost = pl.CostEstimate(flops=flops, transcendentals=0,
                           bytes_accessed=bytes_accessed)

    return pl.pallas_call(
        _gc_kernel,
        out_shape=jax.ShapeDtypeStruct((B, W, N, Fout), nodes.dtype),
        grid_spec=pl.GridSpec(
            grid=(B,),
            in_specs=[
                pl.BlockSpec((pl.Squeezed(), W, N, N), lambda b: (b, 0, 0, 0)),
                pl.BlockSpec((pl.Squeezed(), W, N, Fin), lambda b: (b, 0, 0, 0)),
                pl.BlockSpec((W, Fin, Fout), lambda b: (0, 0, 0)),
            ],
            out_specs=pl.BlockSpec((pl.Squeezed(), W, N, Fout),
                                   lambda b: (b, 0, 0, 0)),
        ),
        compiler_params=pltpu.CompilerParams(
            dimension_semantics=("parallel",),
            vmem_limit_bytes=48 * 1024 * 1024,
        ),
        cost_estimate=cost,
    )(adjacency, nodes, weights)
